# Initial kernel scaffold; baseline (speedup 1.0000x reference)
import functools

import jax
import jax.numpy as jnp
from jax import lax
from jax.experimental import pallas as pl
from jax.experimental.pallas import tpu as pltpu
from jax.experimental.pallas import tpu_sc as plsc

N = 10000
E = 320000
R = 5
DES = 768
D = 128

LANES = 16
NS = 16
NC = 2
CHUNK = 128
CPT = 158
GCH = NS * CPT
EPAD = GCH * CHUNK
SLICES = 4
SLICE_W = D // SLICES
NRPAD = 50048
STRIPE = NRPAD // NS
ZROWS = 782
CNTW = 16
BN = 1000
NRB = BN * R

_mesh = plsc.VectorSubcoreMesh(core_axis_name="c", subcore_axis_name="s")


def _each16(ref, j, ncols, fn):
    for k in range(ncols // LANES):
        sl = pl.ds(k * LANES, LANES)
        ref[j, sl] = fn(ref[j, sl], k)


@functools.partial(
    pl.kernel,
    out_type=(
        jax.ShapeDtypeStruct((GCH, CHUNK), jnp.int32),
        jax.ShapeDtypeStruct((GCH, CHUNK), jnp.int32),
        jax.ShapeDtypeStruct((NRPAD, CNTW), jnp.float32),
    ),
    mesh=_mesh,
    scratch_types=[
        pltpu.VMEM((CPT, CHUNK), jnp.int32),
        pltpu.VMEM((CPT, CHUNK), jnp.int32),
        pltpu.VMEM((CPT, CHUNK), jnp.int32),
        pltpu.VMEM((CHUNK, CNTW), jnp.float32),
        pltpu.VMEM((ZROWS, CNTW), jnp.float32),
        pltpu.VMEM_SHARED((NRPAD, CNTW), jnp.float32),
    ],
)
def _sc_prep(src_hbm, dst_hbm, et_hbm, key_out, src4_out, cnt_out,
             src_v, dst_v, key_v, ones_v, zeros_v, cnt_sp):
    cid = lax.axis_index("c")
    sid = lax.axis_index("s")

    @pl.when(cid == 0)
    def _():
        def fill_ones(i, c):
            ones_v[i, pl.ds(0, LANES)] = jnp.full((LANES,), 1.0, jnp.float32)
            return c
        lax.fori_loop(0, CHUNK, fill_ones, 0)

        def fill_zeros(i, c):
            zeros_v[i, pl.ds(0, LANES)] = jnp.zeros((LANES,), jnp.float32)
            return c
        lax.fori_loop(0, ZROWS, fill_zeros, 0)

        for z in range(STRIPE // ZROWS):
            pltpu.sync_copy(
                zeros_v, cnt_sp.at[pl.ds(sid * STRIPE + z * ZROWS, ZROWS)])

        base = sid * CPT
        pltpu.sync_copy(src_hbm.at[pl.ds(base, CPT)], src_v)
        pltpu.sync_copy(dst_hbm.at[pl.ds(base, CPT)], dst_v)
        pltpu.sync_copy(et_hbm.at[pl.ds(base, CPT)], key_v)

        def compute(j, c):
            _each16(key_v, j, CHUNK,
                    lambda v, k: dst_v[j, pl.ds(k * LANES, LANES)] * R + v)
            _each16(src_v, j, CHUNK, lambda v, k: v * SLICES)
            return c
        lax.fori_loop(0, CPT, compute, 0)

        pltpu.sync_copy(key_v, key_out.at[pl.ds(base, CPT)])
        pltpu.sync_copy(src_v, src4_out.at[pl.ds(base, CPT)])

        plsc.subcore_barrier()

        def count(j, c):
            pltpu.sync_copy(ones_v, cnt_sp.at[key_v.at[j]], add=True)
            return c
        lax.fori_loop(0, CPT, count, 0)

        plsc.subcore_barrier()
        pltpu.sync_copy(cnt_sp.at[pl.ds(sid * STRIPE, STRIPE)],
                        cnt_out.at[pl.ds(sid * STRIPE, STRIPE)])


@functools.partial(
    pl.kernel,
    out_type=jax.ShapeDtypeStruct((SLICES * NRPAD, SLICE_W), jnp.float32),
    mesh=_mesh,
    scratch_types=[
        pltpu.VMEM((CPT, CHUNK), jnp.int32),
        pltpu.VMEM((CPT, CHUNK), jnp.int32),
        pltpu.VMEM((CHUNK, SLICE_W), jnp.float32),
        pltpu.VMEM((ZROWS, SLICE_W), jnp.float32),
        pltpu.VMEM_SHARED((NRPAD, SLICE_W), jnp.float32),
        pltpu.SemaphoreType.DMA,
    ],
)
def _sc_agg(x4_hbm, src4_hbm, key_hbm, agg_out,
            idx_v, key_v, rows_v, zeros_v, agg_sp, sem):
    cid = lax.axis_index("c")
    sid = lax.axis_index("s")

    def fill_zeros(i, c):
        for k in range(SLICE_W // LANES):
            zeros_v[i, pl.ds(k * LANES, LANES)] = jnp.zeros(
                (LANES,), jnp.float32)
        return c
    lax.fori_loop(0, ZROWS, fill_zeros, 0)

    base = sid * CPT
    pltpu.sync_copy(src4_hbm.at[pl.ds(base, CPT)], idx_v)
    pltpu.sync_copy(key_hbm.at[pl.ds(base, CPT)], key_v)

    def add_base(j, c):
        _each16(idx_v, j, CHUNK, lambda v, k: v + 2 * cid)
        return c
    lax.fori_loop(0, CPT, add_base, 0)

    def zero_stripe():
        for z in range(STRIPE // ZROWS):
            pltpu.sync_copy(
                zeros_v, agg_sp.at[pl.ds(sid * STRIPE + z * ZROWS, ZROWS)])

    zero_stripe()
    plsc.subcore_barrier()

    for p in range(SLICES // NC):
        def chunk(j, c):
            pltpu.async_copy(x4_hbm.at[idx_v.at[j]], rows_v, sem).wait()
            pltpu.sync_copy(rows_v, agg_sp.at[key_v.at[j]], add=True)
            return c
        lax.fori_loop(0, CPT, chunk, 0)

        plsc.subcore_barrier()
        slice_idx = 2 * cid + p
        pltpu.sync_copy(
            agg_sp.at[pl.ds(sid * STRIPE, STRIPE)],
            agg_out.at[pl.ds(slice_idx * NRPAD + sid * STRIPE, STRIPE)])

        if p == 0:
            zero_stripe()

            def bump(j, c):
                _each16(idx_v, j, CHUNK, lambda v, k: v + 1)
                return c
            lax.fori_loop(0, CPT, bump, 0)
            plsc.subcore_barrier()


def _leaky(x):
    return jnp.where(x > 0, x, 0.01 * x)


def _mlp_body(des_ref, w1_ref, b1_ref, w2_ref, b2_ref, out_ref):
    h = jnp.dot(des_ref[...], w1_ref[...], preferred_element_type=jnp.float32)
    h = _leaky(h + b1_ref[...])
    h = jnp.dot(h, w2_ref[...], preferred_element_type=jnp.float32)
    out_ref[...] = _leaky(h + b2_ref[...])


def _rgcn_body(final, agg_ref, cnt_ref, x_ref, wrel_ref, wroot_ref, brg_ref,
               w3_ref, b3_ref, w4_ref, b4_ref, out_ref):
    inv = 1.0 / jnp.maximum(cnt_ref[...][:, 0:1], 1.0)
    mean = jnp.concatenate(
        [agg_ref[c] * inv for c in range(SLICES)], axis=1)
    mean = mean.reshape(BN, R * D)
    acc = jnp.dot(x_ref[...], wroot_ref[...],
                  preferred_element_type=jnp.float32) + brg_ref[...]
    for r in range(R):
        acc = acc + jnp.dot(mean[:, r * D:(r + 1) * D], wrel_ref[r],
                            preferred_element_type=jnp.float32)
    if final:
        h = _leaky(jnp.dot(acc, w3_ref[...],
                           preferred_element_type=jnp.float32) + b3_ref[...])
        out_ref[...] = jnp.dot(h, w4_ref[...],
                               preferred_element_type=jnp.float32) + b4_ref[...]
    else:
        out_ref[...] = acc


def _mlp_call(des, w1, b1, w2, b2):
    return pl.pallas_call(
        _mlp_body,
        grid=(N // BN,),
        in_specs=[
            pl.BlockSpec((BN, DES), lambda i: (i, 0)),
            pl.BlockSpec((DES, D), lambda i: (0, 0)),
            pl.BlockSpec((1, D), lambda i: (0, 0)),
            pl.BlockSpec((D, D), lambda i: (0, 0)),
            pl.BlockSpec((1, D), lambda i: (0, 0)),
        ],
        out_specs=pl.BlockSpec((BN, D), lambda i: (i, 0)),
        out_shape=jax.ShapeDtypeStruct((N, D), jnp.float32),
    )(des, w1, b1.reshape(1, D), w2, b2.reshape(1, D))


def _rgcn_call(final, agg, cnt, x, wrel, wroot, brg, w3, b3, w4, b4):
    out_w = 2 if final else D
    return pl.pallas_call(
        functools.partial(_rgcn_body, final),
        grid=(N // BN,),
        in_specs=[
            pl.BlockSpec((SLICES, NRB, SLICE_W), lambda i: (0, i, 0)),
            pl.BlockSpec((NRB, CNTW), lambda i: (i, 0)),
            pl.BlockSpec((BN, D), lambda i: (i, 0)),
            pl.BlockSpec((R, D, D), lambda i: (0, 0, 0)),
            pl.BlockSpec((D, D), lambda i: (0, 0)),
            pl.BlockSpec((1, D), lambda i: (0, 0)),
            pl.BlockSpec((D, D), lambda i: (0, 0)),
            pl.BlockSpec((1, D), lambda i: (0, 0)),
            pl.BlockSpec((D, 2), lambda i: (0, 0)),
            pl.BlockSpec((1, 2), lambda i: (0, 0)),
        ],
        out_specs=pl.BlockSpec((BN, out_w), lambda i: (i, 0)),
        out_shape=jax.ShapeDtypeStruct((N, out_w), jnp.float32),
    )(agg, cnt, x, wrel, wroot, brg.reshape(1, D),
      w3, b3.reshape(1, D), w4, b4.reshape(1, 2))


def kernel(des, tweet, num_prop, cat_prop, edge_index, edge_type,
           W1, b1, W2, b2, Wrel, Wroot, brg, W3, b3, W4, b4):
    src = edge_index[0]
    dst = edge_index[1]
    et = edge_type.astype(jnp.int32)
    pad = EPAD - E
    src_p = jnp.concatenate(
        [src, jnp.zeros((pad,), jnp.int32)]).reshape(GCH, CHUNK)
    dst_p = jnp.concatenate(
        [dst, jnp.full((pad,), N, jnp.int32)]).reshape(GCH, CHUNK)
    et_p = jnp.concatenate(
        [et, jnp.zeros((pad,), jnp.int32)]).reshape(GCH, CHUNK)

    key3, src4, cnt = _sc_prep(src_p, dst_p, et_p)

    x1 = _mlp_call(des, W1, b1, W2, b2)
    agg1 = _sc_agg(x1.reshape(SLICES * N, SLICE_W), src4, key3)
    x2 = _rgcn_call(False, agg1.reshape(SLICES, NRPAD, SLICE_W), cnt, x1,
                    Wrel, Wroot, brg, W3, b3, W4, b4)
    agg2 = _sc_agg(x2.reshape(SLICES * N, SLICE_W), src4, key3)
    out = _rgcn_call(True, agg2.reshape(SLICES, NRPAD, SLICE_W), cnt, x2,
                     Wrel, Wroot, brg, W3, b3, W4, b4)
    return out


# speedup vs baseline: 2.1253x; 2.1253x over previous
import functools

import jax
import jax.numpy as jnp
from jax import lax
from jax.experimental import pallas as pl
from jax.experimental.pallas import tpu as pltpu
from jax.experimental.pallas import tpu_sc as plsc

N = 10000
E = 320000
R = 5
DES = 768
D = 128

LANES = 16
NS = 16
NC = 2
CHUNK = 128
CPT = 160
GCH = NS * CPT
EPAD = GCH * CHUNK
SLICES = 8
SLICE_W = D // SLICES
NRPAD = 50048
STRIPE = NRPAD // NS
ZROWS = 184
CNTW = 8
BN = 400
NRB = BN * R

_mesh = plsc.VectorSubcoreMesh(core_axis_name="c", subcore_axis_name="s")


def _each16(ref, j, ncols, fn):
    for k in range(ncols // LANES):
        sl = pl.ds(k * LANES, LANES)
        ref[j, sl] = fn(ref[j, sl], k)


@functools.partial(
    pl.kernel,
    out_type=(
        jax.ShapeDtypeStruct((GCH, CHUNK), jnp.int32),
        jax.ShapeDtypeStruct((GCH, CHUNK), jnp.int32),
        jax.ShapeDtypeStruct((NRPAD, CNTW), jnp.float32),
    ),
    mesh=_mesh,
    compiler_params=pltpu.CompilerParams(use_tc_tiling_on_sc=False),
    scratch_types=[
        pltpu.VMEM((CPT, CHUNK), jnp.int32),
        pltpu.VMEM((CPT, CHUNK), jnp.int32),
        pltpu.VMEM((CPT, CHUNK), jnp.int32),
        pltpu.VMEM((CHUNK, CNTW), jnp.float32),
        pltpu.VMEM((ZROWS, CNTW), jnp.float32),
        pltpu.VMEM_SHARED((NRPAD, CNTW), jnp.float32),
    ],
)
def _sc_prep(src_hbm, dst_hbm, et_hbm, key_out, src4_out, cnt_out,
             src_v, dst_v, key_v, ones_v, zeros_v, cnt_sp):
    cid = lax.axis_index("c")
    sid = lax.axis_index("s")

    @pl.when(cid == 0)
    def _():
        def fill_ones(i, c):
            ones_v[i, pl.ds(0, LANES)] = jnp.full((LANES,), 1.0, jnp.float32)
            return c
        lax.fori_loop(0, CHUNK, fill_ones, 0)

        def fill_zeros(i, c):
            zeros_v[i, pl.ds(0, LANES)] = jnp.zeros((LANES,), jnp.float32)
            return c
        lax.fori_loop(0, ZROWS, fill_zeros, 0)

        for z in range(STRIPE // ZROWS):
            pltpu.sync_copy(
                zeros_v, cnt_sp.at[pl.ds(sid * STRIPE + z * ZROWS, ZROWS)])

        base = sid * CPT
        pltpu.sync_copy(src_hbm.at[pl.ds(base, CPT)], src_v)
        pltpu.sync_copy(dst_hbm.at[pl.ds(base, CPT)], dst_v)
        pltpu.sync_copy(et_hbm.at[pl.ds(base, CPT)], key_v)

        def compute(j, c):
            _each16(key_v, j, CHUNK,
                    lambda v, k: dst_v[j, pl.ds(k * LANES, LANES)] * R + v)
            _each16(src_v, j, CHUNK, lambda v, k: v * SLICES)
            return c
        lax.fori_loop(0, CPT, compute, 0)

        pltpu.sync_copy(key_v, key_out.at[pl.ds(base, CPT)])
        pltpu.sync_copy(src_v, src4_out.at[pl.ds(base, CPT)])

        plsc.subcore_barrier()

        def count(j, c):
            pltpu.sync_copy(ones_v, cnt_sp.at[key_v.at[j]], add=True)
            return c
        lax.fori_loop(0, CPT, count, 0)

        plsc.subcore_barrier()
        pltpu.sync_copy(cnt_sp.at[pl.ds(sid * STRIPE, STRIPE)],
                        cnt_out.at[pl.ds(sid * STRIPE, STRIPE)])


@functools.partial(
    pl.kernel,
    out_type=jax.ShapeDtypeStruct((SLICES * NRPAD, SLICE_W), jnp.float32),
    mesh=_mesh,
    compiler_params=pltpu.CompilerParams(use_tc_tiling_on_sc=False),
    scratch_types=[
        pltpu.VMEM((CPT, CHUNK), jnp.int32),
        pltpu.VMEM((CPT, CHUNK), jnp.int32),
        pltpu.VMEM((CHUNK, SLICE_W), jnp.float32),
        pltpu.VMEM((ZROWS, SLICE_W), jnp.float32),
        pltpu.VMEM_SHARED((NRPAD, SLICE_W), jnp.float32),
        pltpu.SemaphoreType.DMA,
    ],
)
def _sc_agg(x4_hbm, src4_hbm, key_hbm, agg_out,
            idx_v, key_v, rows_v, zeros_v, agg_sp, sem):
    cid = lax.axis_index("c")
    sid = lax.axis_index("s")

    def fill_zeros(i, c):
        for k in range(SLICE_W // LANES):
            zeros_v[i, pl.ds(k * LANES, LANES)] = jnp.zeros(
                (LANES,), jnp.float32)
        return c
    lax.fori_loop(0, ZROWS, fill_zeros, 0)

    base = sid * CPT
    pltpu.sync_copy(src4_hbm.at[pl.ds(base, CPT)], idx_v)
    pltpu.sync_copy(key_hbm.at[pl.ds(base, CPT)], key_v)

    def add_base(j, c):
        _each16(idx_v, j, CHUNK, lambda v, k: v + (SLICES // NC) * cid)
        return c
    lax.fori_loop(0, CPT, add_base, 0)

    def zero_stripe():
        for z in range(STRIPE // ZROWS):
            pltpu.sync_copy(
                zeros_v, agg_sp.at[pl.ds(sid * STRIPE + z * ZROWS, ZROWS)])

    zero_stripe()
    plsc.subcore_barrier()

    for p in range(SLICES // NC):
        def chunk(j, c):
            pltpu.async_copy(x4_hbm.at[idx_v.at[j]], rows_v, sem).wait()
            pltpu.sync_copy(rows_v, agg_sp.at[key_v.at[j]], add=True)
            return c
        lax.fori_loop(0, CPT, chunk, 0)

        plsc.subcore_barrier()
        slice_idx = (SLICES // NC) * cid + p
        pltpu.sync_copy(
            agg_sp.at[pl.ds(sid * STRIPE, STRIPE)],
            agg_out.at[pl.ds(slice_idx * NRPAD + sid * STRIPE, STRIPE)])

        if p < SLICES // NC - 1:
            zero_stripe()

            def bump(j, c):
                _each16(idx_v, j, CHUNK, lambda v, k: v + 1)
                return c
            lax.fori_loop(0, CPT, bump, 0)
            plsc.subcore_barrier()


def _leaky(x):
    return jnp.where(x > 0, x, 0.01 * x)


def _mlp_body(des_ref, w1_ref, b1_ref, w2_ref, b2_ref, out_ref):
    h = jnp.dot(des_ref[...], w1_ref[...], preferred_element_type=jnp.float32)
    h = _leaky(h + b1_ref[...])
    h = jnp.dot(h, w2_ref[...], preferred_element_type=jnp.float32)
    out_ref[...] = _leaky(h + b2_ref[...])


def _rgcn_body(final, agg_ref, cnt_ref, x_ref, wrel_ref, wroot_ref, brg_ref,
               w3_ref, b3_ref, w4_ref, b4_ref, out_ref):
    inv = 1.0 / jnp.maximum(cnt_ref[...][:, 0:1], 1.0)
    mean = jnp.concatenate(
        [agg_ref[c] * inv for c in range(SLICES)], axis=1)
    mean = mean.reshape(BN, R * D)
    acc = jnp.dot(x_ref[...], wroot_ref[...],
                  preferred_element_type=jnp.float32) + brg_ref[...]
    for r in range(R):
        acc = acc + jnp.dot(mean[:, r * D:(r + 1) * D], wrel_ref[r],
                            preferred_element_type=jnp.float32)
    if final:
        h = _leaky(jnp.dot(acc, w3_ref[...],
                           preferred_element_type=jnp.float32) + b3_ref[...])
        out_ref[...] = jnp.dot(h, w4_ref[...],
                               preferred_element_type=jnp.float32) + b4_ref[...]
    else:
        out_ref[...] = acc


def _mlp_call(des, w1, b1, w2, b2):
    return pl.pallas_call(
        _mlp_body,
        grid=(N // BN,),
        in_specs=[
            pl.BlockSpec((BN, DES), lambda i: (i, 0)),
            pl.BlockSpec((DES, D), lambda i: (0, 0)),
            pl.BlockSpec((1, D), lambda i: (0, 0)),
            pl.BlockSpec((D, D), lambda i: (0, 0)),
            pl.BlockSpec((1, D), lambda i: (0, 0)),
        ],
        out_specs=pl.BlockSpec((BN, D), lambda i: (i, 0)),
        out_shape=jax.ShapeDtypeStruct((N, D), jnp.float32),
    )(des, w1, b1.reshape(1, D), w2, b2.reshape(1, D))


def _rgcn_call(final, agg, cnt, x, wrel, wroot, brg, w3, b3, w4, b4):
    out_w = 2 if final else D
    return pl.pallas_call(
        functools.partial(_rgcn_body, final),
        grid=(N // BN,),
        in_specs=[
            pl.BlockSpec((SLICES, NRB, SLICE_W), lambda i: (0, i, 0)),
            pl.BlockSpec((NRB, CNTW), lambda i: (i, 0)),
            pl.BlockSpec((BN, D), lambda i: (i, 0)),
            pl.BlockSpec((R, D, D), lambda i: (0, 0, 0)),
            pl.BlockSpec((D, D), lambda i: (0, 0)),
            pl.BlockSpec((1, D), lambda i: (0, 0)),
            pl.BlockSpec((D, D), lambda i: (0, 0)),
            pl.BlockSpec((1, D), lambda i: (0, 0)),
            pl.BlockSpec((D, 2), lambda i: (0, 0)),
            pl.BlockSpec((1, 2), lambda i: (0, 0)),
        ],
        out_specs=pl.BlockSpec((BN, out_w), lambda i: (i, 0)),
        out_shape=jax.ShapeDtypeStruct((N, out_w), jnp.float32),
    )(agg, cnt, x, wrel, wroot, brg.reshape(1, D),
      w3, b3.reshape(1, D), w4, b4.reshape(1, 2))


def kernel(des, tweet, num_prop, cat_prop, edge_index, edge_type,
           W1, b1, W2, b2, Wrel, Wroot, brg, W3, b3, W4, b4):
    src = edge_index[0]
    dst = edge_index[1]
    et = edge_type.astype(jnp.int32)
    pad = EPAD - E
    src_p = jnp.concatenate(
        [src, jnp.zeros((pad,), jnp.int32)]).reshape(GCH, CHUNK)
    dst_p = jnp.concatenate(
        [dst, jnp.full((pad,), N, jnp.int32)]).reshape(GCH, CHUNK)
    et_p = jnp.concatenate(
        [et, jnp.zeros((pad,), jnp.int32)]).reshape(GCH, CHUNK)

    key3, src4, cnt = _sc_prep(src_p, dst_p, et_p)

    x1 = _mlp_call(des, W1, b1, W2, b2)
    agg1 = _sc_agg(x1.reshape(SLICES * N, SLICE_W), src4, key3)
    x2 = _rgcn_call(False, agg1.reshape(SLICES, NRPAD, SLICE_W), cnt, x1,
                    Wrel, Wroot, brg, W3, b3, W4, b4)
    agg2 = _sc_agg(x2.reshape(SLICES * N, SLICE_W), src4, key3)
    out = _rgcn_call(True, agg2.reshape(SLICES, NRPAD, SLICE_W), cnt, x2,
                     Wrel, Wroot, brg, W3, b3, W4, b4)
    return out


# speedup vs baseline: 3.1576x; 1.4857x over previous
import functools

import jax
import jax.numpy as jnp
from jax import lax
from jax.experimental import pallas as pl
from jax.experimental.pallas import tpu as pltpu
from jax.experimental.pallas import tpu_sc as plsc

N = 10000
E = 320000
R = 5
DES = 768
D = 128

LANES = 16
NS = 16
NC = 2
CHUNK = 128
CPT = 160
GCH = NS * CPT
EPAD = GCH * CHUNK
SLICES = 8
SLICE_W = D // SLICES
NRPAD = 50048
STRIPE = NRPAD // NS
ZROWS = 184
CNTW = 8
SETN = 4
BN = 400
NRB = BN * R

_mesh = plsc.VectorSubcoreMesh(core_axis_name="c", subcore_axis_name="s")


def _each16(ref, j, ncols, fn):
    for k in range(ncols // LANES):
        sl = pl.ds(k * LANES, LANES)
        ref[j, sl] = fn(ref[j, sl], k)


@functools.partial(
    pl.kernel,
    out_type=(
        jax.ShapeDtypeStruct((GCH, CHUNK), jnp.int32),
        jax.ShapeDtypeStruct((GCH, CHUNK), jnp.int32),
        jax.ShapeDtypeStruct((NRPAD, CNTW), jnp.float32),
    ),
    mesh=_mesh,
    compiler_params=pltpu.CompilerParams(use_tc_tiling_on_sc=False),
    scratch_types=[
        pltpu.VMEM((CPT, CHUNK), jnp.int32),
        pltpu.VMEM((CPT, CHUNK), jnp.int32),
        pltpu.VMEM((CPT, CHUNK), jnp.int32),
        pltpu.VMEM((CHUNK, CNTW), jnp.float32),
        pltpu.VMEM((ZROWS, CNTW), jnp.float32),
        pltpu.VMEM_SHARED((NRPAD, CNTW), jnp.float32),
    ],
)
def _sc_prep(src_hbm, dst_hbm, et_hbm, key_out, src4_out, cnt_out,
             src_v, dst_v, key_v, ones_v, zeros_v, cnt_sp):
    cid = lax.axis_index("c")
    sid = lax.axis_index("s")

    @pl.when(cid == 0)
    def _():
        def fill_ones(i, c):
            ones_v[i, pl.ds(0, LANES)] = jnp.full((LANES,), 1.0, jnp.float32)
            return c
        lax.fori_loop(0, CHUNK, fill_ones, 0)

        def fill_zeros(i, c):
            zeros_v[i, pl.ds(0, LANES)] = jnp.zeros((LANES,), jnp.float32)
            return c
        lax.fori_loop(0, ZROWS, fill_zeros, 0)

        for z in range(STRIPE // ZROWS):
            pltpu.sync_copy(
                zeros_v, cnt_sp.at[pl.ds(sid * STRIPE + z * ZROWS, ZROWS)])

        base = sid * CPT
        pltpu.sync_copy(src_hbm.at[pl.ds(base, CPT)], src_v)
        pltpu.sync_copy(dst_hbm.at[pl.ds(base, CPT)], dst_v)
        pltpu.sync_copy(et_hbm.at[pl.ds(base, CPT)], key_v)

        def compute(j, c):
            _each16(key_v, j, CHUNK,
                    lambda v, k: dst_v[j, pl.ds(k * LANES, LANES)] * R + v)
            _each16(src_v, j, CHUNK, lambda v, k: v * SLICES)
            return c
        lax.fori_loop(0, CPT, compute, 0)

        pltpu.sync_copy(key_v, key_out.at[pl.ds(base, CPT)])
        pltpu.sync_copy(src_v, src4_out.at[pl.ds(base, CPT)])

        plsc.subcore_barrier()

        def count(j, c):
            pltpu.sync_copy(ones_v, cnt_sp.at[key_v.at[j]], add=True)
            return c
        lax.fori_loop(0, CPT, count, 0)

        plsc.subcore_barrier()
        pltpu.sync_copy(cnt_sp.at[pl.ds(sid * STRIPE, STRIPE)],
                        cnt_out.at[pl.ds(sid * STRIPE, STRIPE)])


@functools.partial(
    pl.kernel,
    out_type=jax.ShapeDtypeStruct((SLICES, NRPAD, SLICE_W), jnp.float32),
    mesh=_mesh,
    compiler_params=pltpu.CompilerParams(use_tc_tiling_on_sc=False),
    scratch_types=[
        pltpu.VMEM((CPT, CHUNK), jnp.int32),
        pltpu.VMEM((CPT, CHUNK), jnp.int32),
        [[pltpu.VMEM((CHUNK, SLICE_W), jnp.float32) for _ in range(SETN)]
         for _ in range(2)],
        pltpu.VMEM((ZROWS, SLICE_W), jnp.float32),
        pltpu.VMEM_SHARED((NRPAD, SLICE_W), jnp.float32),
        [[pltpu.SemaphoreType.DMA for _ in range(SETN)] for _ in range(2)],
        [[pltpu.SemaphoreType.DMA for _ in range(SETN)] for _ in range(2)],
    ],
)
def _sc_agg(x4_hbm, src4_hbm, key_hbm, agg_out,
            idx_v, key_v, rows_v, zeros_v, agg_sp, gsem, ssem):
    cid = lax.axis_index("c")
    sid = lax.axis_index("s")

    def fill_zeros(i, c):
        for k in range(SLICE_W // LANES):
            zeros_v[i, pl.ds(k * LANES, LANES)] = jnp.zeros(
                (LANES,), jnp.float32)
        return c
    lax.fori_loop(0, ZROWS, fill_zeros, 0)

    base = sid * CPT
    pltpu.sync_copy(src4_hbm.at[pl.ds(base, CPT)], idx_v)
    pltpu.sync_copy(key_hbm.at[pl.ds(base, CPT)], key_v)

    def add_base(j, c):
        _each16(idx_v, j, CHUNK, lambda v, k: v + (SLICES // NC) * cid)
        return c
    lax.fori_loop(0, CPT, add_base, 0)

    def zero_stripe():
        for z in range(STRIPE // ZROWS):
            pltpu.sync_copy(
                zeros_v, agg_sp.at[pl.ds(sid * STRIPE + z * ZROWS, ZROWS)])

    zero_stripe()
    plsc.subcore_barrier()

    NG = CPT // SETN
    for p in range(SLICES // NC):
        for s in range(2):
            for b in range(SETN):
                pltpu.async_copy(
                    x4_hbm.at[idx_v.at[s * SETN + b]], rows_v[s][b],
                    gsem[s][b])

        def pair(gp, c):
            for s in range(2):
                g = 2 * gp + s
                gbase = g * SETN
                for b in range(SETN):
                    j = gbase + b
                    pltpu.make_async_copy(
                        x4_hbm.at[idx_v.at[j]], rows_v[s][b],
                        gsem[s][b]).wait()
                    pltpu.async_copy(rows_v[s][b], agg_sp.at[key_v.at[j]],
                                     ssem[s][b], add=True)
                for b in range(SETN):
                    j = gbase + b
                    pltpu.make_async_copy(
                        rows_v[s][b], agg_sp.at[key_v.at[j]],
                        ssem[s][b]).wait()

                    @pl.when(g + 2 < NG)
                    def _():
                        pltpu.async_copy(
                            x4_hbm.at[idx_v.at[j + 2 * SETN]], rows_v[s][b],
                            gsem[s][b])
            return c
        lax.fori_loop(0, NG // 2, pair, 0)

        plsc.subcore_barrier()
        slice_idx = (SLICES // NC) * cid + p
        pltpu.sync_copy(
            agg_sp.at[pl.ds(sid * STRIPE, STRIPE)],
            agg_out.at[slice_idx, pl.ds(sid * STRIPE, STRIPE)])

        if p < SLICES // NC - 1:
            zero_stripe()

            def bump(j, c):
                _each16(idx_v, j, CHUNK, lambda v, k: v + 1)
                return c
            lax.fori_loop(0, CPT, bump, 0)
            plsc.subcore_barrier()


def _leaky(x):
    return jnp.where(x > 0, x, 0.01 * x)


def _mlp_body(des_ref, w1_ref, b1_ref, w2_ref, b2_ref, out_ref):
    h = jnp.dot(des_ref[...], w1_ref[...], preferred_element_type=jnp.float32)
    h = _leaky(h + b1_ref[...])
    h = jnp.dot(h, w2_ref[...], preferred_element_type=jnp.float32)
    out_ref[...] = _leaky(h + b2_ref[...])


def _rgcn_body(final, agg_ref, cnt_ref, x_ref, wrel_ref, wroot_ref, brg_ref,
               w3_ref, b3_ref, w4_ref, b4_ref, out_ref):
    inv = 1.0 / jnp.maximum(cnt_ref[...][:, 0:1], 1.0)
    mean = agg_ref[...] * inv
    mean = mean.reshape(BN, R * D)
    acc = jnp.dot(x_ref[...], wroot_ref[...],
                  preferred_element_type=jnp.float32) + brg_ref[...]
    for r in range(R):
        acc = acc + jnp.dot(mean[:, r * D:(r + 1) * D], wrel_ref[r],
                            preferred_element_type=jnp.float32)
    if final:
        h = _leaky(jnp.dot(acc, w3_ref[...],
                           preferred_element_type=jnp.float32) + b3_ref[...])
        out_ref[...] = jnp.dot(h, w4_ref[...],
                               preferred_element_type=jnp.float32) + b4_ref[...]
    else:
        out_ref[...] = acc


def _mlp_call(des, w1, b1, w2, b2):
    return pl.pallas_call(
        _mlp_body,
        grid=(N // BN,),
        in_specs=[
            pl.BlockSpec((BN, DES), lambda i: (i, 0)),
            pl.BlockSpec((DES, D), lambda i: (0, 0)),
            pl.BlockSpec((1, D), lambda i: (0, 0)),
            pl.BlockSpec((D, D), lambda i: (0, 0)),
            pl.BlockSpec((1, D), lambda i: (0, 0)),
        ],
        out_specs=pl.BlockSpec((BN, D), lambda i: (i, 0)),
        out_shape=jax.ShapeDtypeStruct((N, D), jnp.float32),
    )(des, w1, b1.reshape(1, D), w2, b2.reshape(1, D))


def _rgcn_call(final, agg, cnt, x, wrel, wroot, brg, w3, b3, w4, b4):
    out_w = 2 if final else D
    return pl.pallas_call(
        functools.partial(_rgcn_body, final),
        grid=(N // BN,),
        in_specs=[
            pl.BlockSpec((NRB, D), lambda i: (i, 0)),
            pl.BlockSpec((NRB, CNTW), lambda i: (i, 0)),
            pl.BlockSpec((BN, D), lambda i: (i, 0)),
            pl.BlockSpec((R, D, D), lambda i: (0, 0, 0)),
            pl.BlockSpec((D, D), lambda i: (0, 0)),
            pl.BlockSpec((1, D), lambda i: (0, 0)),
            pl.BlockSpec((D, D), lambda i: (0, 0)),
            pl.BlockSpec((1, D), lambda i: (0, 0)),
            pl.BlockSpec((D, 2), lambda i: (0, 0)),
            pl.BlockSpec((1, 2), lambda i: (0, 0)),
        ],
        out_specs=pl.BlockSpec((BN, out_w), lambda i: (i, 0)),
        out_shape=jax.ShapeDtypeStruct((N, out_w), jnp.float32),
    )(agg, cnt, x, wrel, wroot, brg.reshape(1, D),
      w3, b3.reshape(1, D), w4, b4.reshape(1, 2))


def kernel(des, tweet, num_prop, cat_prop, edge_index, edge_type,
           W1, b1, W2, b2, Wrel, Wroot, brg, W3, b3, W4, b4):
    src = edge_index[0]
    dst = edge_index[1]
    et = edge_type.astype(jnp.int32)
    pad = EPAD - E
    src_p = jnp.concatenate(
        [src, jnp.zeros((pad,), jnp.int32)]).reshape(GCH, CHUNK)
    dst_p = jnp.concatenate(
        [dst, jnp.full((pad,), N, jnp.int32)]).reshape(GCH, CHUNK)
    et_p = jnp.concatenate(
        [et, jnp.zeros((pad,), jnp.int32)]).reshape(GCH, CHUNK)

    key3, src4, cnt = _sc_prep(src_p, dst_p, et_p)

    x1 = _mlp_call(des, W1, b1, W2, b2)
    agg1 = _sc_agg(x1.reshape(SLICES * N, SLICE_W), src4, key3)
    mean1 = agg1.transpose(1, 0, 2).reshape(NRPAD, D)
    x2 = _rgcn_call(False, mean1, cnt, x1,
                    Wrel, Wroot, brg, W3, b3, W4, b4)
    agg2 = _sc_agg(x2.reshape(SLICES * N, SLICE_W), src4, key3)
    mean2 = agg2.transpose(1, 0, 2).reshape(NRPAD, D)
    out = _rgcn_call(True, mean2, cnt, x2,
                     Wrel, Wroot, brg, W3, b3, W4, b4)
    return out
